# baseline (speedup 1.0000x reference)
import numpy as np
import jax
import jax.numpy as jnp
from jax import lax
from jax.experimental import pallas as pl
from jax.experimental.pallas import tpu as pltpu
from jax.experimental.pallas import tpu_sc as plsc

_B, _C, _H, _W = 8, 4, 512, 512
_HW = _H * _W
_NPIX = _B * _HW
_NW = 32
_PPW = _NPIX // _NW
_T = 8192
_NVEC = _T // 16
_K_SC = 1
_STEPS = _K_SC

_ALPHA = 0.75
_W4 = np.array([1 - 0.694139, 1 - 0.088105, 1 - 0.072427, 1 - 0.145329],
               dtype=np.float32)
_WTAB = -(_ALPHA * np.log(2.0) / _NPIX) * _W4.astype(np.float64)

_LO = np.float32(1e-8)

_P4, _P3, _P2, _P1 = -0.08001088, 0.3154676, -0.67293417, 1.4373021
_P0M127 = 0.00010018903 - 127.0
_Q0, _Q1, _Q2, _Q3 = (-7.58195e-08, -2.6486185e-07, 1.3536362e-07,
                      -2.0730770e-08)


def _sc_body(out_hbm, mask_hbm, res_hbm,
             xb0, xb1, mb0, mb1, accb, sem0, sem1):
    ci = lax.axis_index("c")
    si = lax.axis_index("s")
    wid = si * 2 + ci
    xbufs = (xb0, xb1)
    mbufs = (mb0, mb1)
    sems = (sem0, sem1)

    def issue(s, p):
        g = wid * _K_SC + s
        b = (_B - _K_SC) + g // 32
        h0 = (g % 32) * 16
        ds = []
        ds.append(pltpu.async_copy(
            mask_hbm.at[b, 0, pl.ds(h0, 16), :], mbufs[p], sems[p]))
        for c in range(_C):
            src = out_hbm.at[b, c, pl.ds(h0, 16), :]
            ds.append(pltpu.async_copy(
                src, xbufs[p].at[pl.ds(c * 16, 16), :], sems[p]))
        return ds

    iot = lax.iota(jnp.int32, 16)
    zero = jnp.zeros((16,), jnp.float32)
    accs = (zero, zero, zero, zero)
    pend = issue(0, 0)
    for s in range(_STEPS):
        p = s & 1
        for d in pend:
            d.wait()
        if s + 1 < _STEPS:
            pend = issue(s + 1, p ^ 1)
        xb = xbufs[p]
        mb = mbufs[p]

        def body(v, accs, xb=xb, mb=mb):
            out = []
            for j in range(4):
                vj = v + j
                r = vj & 15
                w0 = vj & -16
                w_idx = iot + w0
                m = mb[r, pl.ds(w0, 16)]
                x = plsc.load_gather(xb, [(m << 4) + r, w_idx])
                u = plsc.bitcast(jnp.maximum(x, _LO), jnp.int32)
                ef = (u >> 23).astype(jnp.float32)
                mf = plsc.bitcast((u & 0x007FFFFF) | 0x3F800000,
                                  jnp.float32)
                rm = mf - 1.0
                pv = _P4
                for co in (_P3, _P2, _P1, _P0M127):
                    pv = pv * rm + co
                l2 = ef + pv
                mfl = m.astype(jnp.float32)
                wv = _Q3
                for co in (_Q2, _Q1, _Q0):
                    wv = wv * mfl + co
                out.append(accs[j] + l2 * wv)
            return tuple(out)

        accs = plsc.parallel_loop(0, _NVEC, step=4, unroll=1,
                                  carry=accs)(body)

    accb[...] = (accs[0] + accs[1]) + (accs[2] + accs[3])
    pltpu.sync_copy(accb, res_hbm.at[pl.ds(wid * 16, 16)])


_sc_loss = pl.kernel(
    _sc_body,
    out_type=jax.ShapeDtypeStruct((_NW * 16,), jnp.float32),
    mesh=plsc.VectorSubcoreMesh(core_axis_name="c", subcore_axis_name="s"),
    compiler_params=pltpu.CompilerParams(needs_layout_passes=False,
                                         use_tc_tiling_on_sc=True),
    scratch_types=[
        pltpu.VMEM((_C * 16, 512), jnp.float32),
        pltpu.VMEM((_C * 16, 512), jnp.float32),
        pltpu.VMEM((16, 512), jnp.int32),
        pltpu.VMEM((16, 512), jnp.int32),
        pltpu.VMEM((16,), jnp.float32),
        pltpu.SemaphoreType.DMA,
        pltpu.SemaphoreType.DMA,
    ],
)


_HI = np.float32(1.0 - 1e-8)
_SCALE = np.float32(-_ALPHA / _NPIX)
_W0, _W1, _W2, _W3 = (np.float32(w) for w in _W4)


def _tc_body(out_ref, mask_ref, part_ref):
    x = out_ref[0]
    m = mask_ref[0, 0]
    e1 = m == 1
    e2 = m == 2
    e3 = m == 3
    sel = jnp.where(e1, x[1], x[0])
    sel = jnp.where(e2, x[2], sel)
    sel = jnp.where(e3, x[3], sel)
    sel = jnp.minimum(jnp.maximum(sel, _LO), _HI)
    w = jnp.where(e1, _W1, _W0)
    w = jnp.where(e2, _W2, w)
    w = jnp.where(e3, _W3, w)
    t = jnp.log(sel) * w
    part_ref[0, 0] = jnp.sum(t.reshape(32, 8, 512), axis=0) * _SCALE


def _tc_loss(output, mask):
    nb = output.shape[0]
    return pl.pallas_call(
        _tc_body,
        grid=(nb, 2),
        in_specs=[
            pl.BlockSpec((1, 4, 256, 512), lambda b, h: (b, 0, h, 0)),
            pl.BlockSpec((1, 1, 256, 512), lambda b, h: (b, 0, h, 0)),
        ],
        out_specs=pl.BlockSpec((1, 1, 8, 512), lambda b, h: (b, h, 0, 0)),
        out_shape=jax.ShapeDtypeStruct((nb, 2, 8, 512), jnp.float32),
    )(output, mask)


def _tc_loss_front(output, mask):
    nb = _B - _K_SC
    return pl.pallas_call(
        _tc_body,
        grid=(nb, 2),
        in_specs=[
            pl.BlockSpec((1, 4, 256, 512), lambda b, h: (b, 0, h, 0)),
            pl.BlockSpec((1, 1, 256, 512), lambda b, h: (b, 0, h, 0)),
        ],
        out_specs=pl.BlockSpec((1, 1, 8, 512), lambda b, h: (b, h, 0, 0)),
        out_shape=jax.ShapeDtypeStruct((nb, 2, 8, 512), jnp.float32),
    )(output, mask)


def _tc_loss_full(output, mask):
    nb = output.shape[0]
    return pl.pallas_call(
        _tc_body_full,
        grid=(nb // 4,),
        in_specs=[
            pl.BlockSpec((4, 4, 512, 512), lambda b: (b, 0, 0, 0)),
            pl.BlockSpec((4, 1, 512, 512), lambda b: (b, 0, 0, 0)),
        ],
        out_specs=pl.BlockSpec((1, 8, 512), lambda b: (b, 0, 0)),
        out_shape=jax.ShapeDtypeStruct((nb // 4, 8, 512), jnp.float32),
    )(output, mask)


def _tc_body_full(out_ref, mask_ref, part_ref):
    acc = None
    for i in range(4):
        x = out_ref[i]
        m = mask_ref[i, 0]
        e1 = m == 1
        e2 = m == 2
        e3 = m == 3
        sel = jnp.where(e1, x[1], x[0])
        sel = jnp.where(e2, x[2], sel)
        sel = jnp.where(e3, x[3], sel)
        sel = jnp.minimum(jnp.maximum(sel, _LO), _HI)
        w = jnp.where(e1, _W1, _W0)
        w = jnp.where(e2, _W2, w)
        w = jnp.where(e3, _W3, w)
        t = jnp.sum((jnp.log(sel) * w).reshape(64, 8, 512), axis=0)
        acc = t if acc is None else acc + t
    part_ref[0] = acc * _SCALE


def kernel(output, mask):
    parts = _tc_loss_full(output, mask)
    return jnp.sum(parts)


# speedup vs baseline: 1.1078x; 1.1078x over previous
import numpy as np
import jax
import jax.numpy as jnp
from jax import lax
from jax.experimental import pallas as pl
from jax.experimental.pallas import tpu as pltpu
from jax.experimental.pallas import tpu_sc as plsc

_B, _C, _H, _W = 8, 4, 512, 512
_HW = _H * _W
_NPIX = _B * _HW
_NW = 32
_NVEC = 8192 // 16
_K_SC = 1
_STEPS = _K_SC

_ALPHA = 0.75
_W4 = np.array([1 - 0.694139, 1 - 0.088105, 1 - 0.072427, 1 - 0.145329],
               dtype=np.float32)

_LO = np.float32(1e-8)
_HI = np.float32(1.0 - 1e-8)
_SCALE = np.float32(-_ALPHA / _NPIX)
_W0, _W1, _W2, _W3 = (np.float32(w) for w in _W4)


def _tc_body(out_ref, mask_ref, part_ref):
    acc = None
    for i in range(2):
        x = out_ref[i]
        m = mask_ref[i, 0]
        e1 = m == 1
        e2 = m == 2
        e3 = m == 3
        sel = jnp.where(e1, x[1], x[0])
        sel = jnp.where(e2, x[2], sel)
        sel = jnp.where(e3, x[3], sel)
        sel = jnp.minimum(jnp.maximum(sel, _LO), _HI)
        w = jnp.where(e1, _W1, _W0)
        w = jnp.where(e2, _W2, w)
        w = jnp.where(e3, _W3, w)
        t = jnp.sum((jnp.log(sel) * w).reshape(64, 8, 512), axis=0)
        acc = t if acc is None else acc + t
    part_ref[0] = acc * _SCALE


def _tc_loss(output, mask):
    return pl.pallas_call(
        _tc_body,
        grid=(_B // 2,),
        in_specs=[
            pl.BlockSpec((2, 4, 512, 512), lambda b: (b, 0, 0, 0)),
            pl.BlockSpec((2, 1, 512, 512), lambda b: (b, 0, 0, 0)),
        ],
        out_specs=pl.BlockSpec((1, 8, 512), lambda b: (b, 0, 0)),
        out_shape=jax.ShapeDtypeStruct((_B // 2, 8, 512), jnp.float32),
    )(output, mask)


def kernel(output, mask):
    parts = _tc_loss(output, mask)
    return jnp.sum(parts)


_P4, _P3, _P2, _P1 = -0.08001088, 0.3154676, -0.67293417, 1.4373021
_P0M127 = 0.00010018903 - 127.0
_Q0, _Q1, _Q2, _Q3 = (-7.58195e-08, -2.6486185e-07, 1.3536362e-07,
                      -2.0730770e-08)


def _sc_body(out_hbm, mask_hbm, res_hbm,
             xb0, xb1, mb0, mb1, accb, sem0, sem1):
    ci = lax.axis_index("c")
    si = lax.axis_index("s")
    wid = si * 2 + ci
    xbufs = (xb0, xb1)
    mbufs = (mb0, mb1)
    sems = (sem0, sem1)

    def issue(s, p):
        g = wid * _K_SC + s
        b = (_B - _K_SC) + g // 32
        h0 = (g % 32) * 16
        ds = []
        ds.append(pltpu.async_copy(
            mask_hbm.at[b, 0, pl.ds(h0, 16), :], mbufs[p], sems[p]))
        for c in range(_C):
            src = out_hbm.at[b, c, pl.ds(h0, 16), :]
            ds.append(pltpu.async_copy(
                src, xbufs[p].at[pl.ds(c * 16, 16), :], sems[p]))
        return ds

    iot = lax.iota(jnp.int32, 16)
    zero = jnp.zeros((16,), jnp.float32)
    accs = (zero, zero, zero, zero)
    pend = issue(0, 0)
    for s in range(_STEPS):
        p = s & 1
        for d in pend:
            d.wait()
        if s + 1 < _STEPS:
            pend = issue(s + 1, p ^ 1)
        xb = xbufs[p]
        mb = mbufs[p]

        def body(v, accs, xb=xb, mb=mb):
            out = []
            for j in range(4):
                vj = v + j
                r = vj & 15
                w0 = vj & -16
                w_idx = iot + w0
                m = mb[r, pl.ds(w0, 16)]
                x = plsc.load_gather(xb, [(m << 4) + r, w_idx])
                u = plsc.bitcast(jnp.maximum(x, _LO), jnp.int32)
                ef = (u >> 23).astype(jnp.float32)
                mf = plsc.bitcast((u & 0x007FFFFF) | 0x3F800000,
                                  jnp.float32)
                rm = mf - 1.0
                pv = _P4
                for co in (_P3, _P2, _P1, _P0M127):
                    pv = pv * rm + co
                l2 = ef + pv
                mfl = m.astype(jnp.float32)
                wv = _Q3
                for co in (_Q2, _Q1, _Q0):
                    wv = wv * mfl + co
                out.append(accs[j] + l2 * wv)
            return tuple(out)

        accs = plsc.parallel_loop(0, _NVEC, step=4, unroll=1,
                                  carry=accs)(body)

    accb[...] = (accs[0] + accs[1]) + (accs[2] + accs[3])
    pltpu.sync_copy(accb, res_hbm.at[pl.ds(wid * 16, 16)])


_sc_loss = pl.kernel(
    _sc_body,
    out_type=jax.ShapeDtypeStruct((_NW * 16,), jnp.float32),
    mesh=plsc.VectorSubcoreMesh(core_axis_name="c", subcore_axis_name="s"),
    compiler_params=pltpu.CompilerParams(needs_layout_passes=False,
                                         use_tc_tiling_on_sc=True),
    scratch_types=[
        pltpu.VMEM((_C * 16, 512), jnp.float32),
        pltpu.VMEM((_C * 16, 512), jnp.float32),
        pltpu.VMEM((16, 512), jnp.int32),
        pltpu.VMEM((16, 512), jnp.int32),
        pltpu.VMEM((16,), jnp.float32),
        pltpu.SemaphoreType.DMA,
        pltpu.SemaphoreType.DMA,
    ],
)


# speedup vs baseline: 1.1140x; 1.0056x over previous
import numpy as np
import jax
import jax.numpy as jnp
from jax import lax
from jax.experimental import pallas as pl
from jax.experimental.pallas import tpu as pltpu
from jax.experimental.pallas import tpu_sc as plsc

_B, _C, _H, _W = 8, 4, 512, 512
_HW = _H * _W
_NPIX = _B * _HW
_NW = 32
_NVEC = 8192 // 16
_K_SC = 1
_STEPS = _K_SC

_ALPHA = 0.75
_W4 = np.array([1 - 0.694139, 1 - 0.088105, 1 - 0.072427, 1 - 0.145329],
               dtype=np.float32)

_LO = np.float32(1e-8)
_HI = np.float32(1.0 - 1e-8)
_SCALE = np.float32(-_ALPHA / _NPIX)
_W0, _W1, _W2, _W3 = (np.float32(w) for w in _W4)


def _tc_body(out_ref, mask_ref, loss_ref, acc_ref):
    b = pl.program_id(0)
    acc = None
    for i in range(2):
        x = out_ref[i]
        m = mask_ref[i, 0]
        e1 = m == 1
        e2 = m == 2
        e3 = m == 3
        sel = jnp.where(e1, x[1], x[0])
        sel = jnp.where(e2, x[2], sel)
        sel = jnp.where(e3, x[3], sel)
        sel = jnp.minimum(jnp.maximum(sel, _LO), _HI)
        w = jnp.where(e1, _W1, _W0)
        w = jnp.where(e2, _W2, w)
        w = jnp.where(e3, _W3, w)
        t = jnp.sum((jnp.log(sel) * w).reshape(64, 8, 512), axis=0)
        acc = t if acc is None else acc + t

    @pl.when(b == 0)
    def _():
        acc_ref[...] = acc

    @pl.when(b > 0)
    def _():
        acc_ref[...] += acc

    @pl.when(b == _B // 2 - 1)
    def _():
        loss_ref[0, 0] = jnp.sum(acc_ref[...]) * _SCALE


def _tc_loss(output, mask):
    return pl.pallas_call(
        _tc_body,
        grid=(_B // 2,),
        in_specs=[
            pl.BlockSpec((2, 4, 512, 512), lambda b: (b, 0, 0, 0)),
            pl.BlockSpec((2, 1, 512, 512), lambda b: (b, 0, 0, 0)),
        ],
        out_specs=pl.BlockSpec((1, 1), lambda b: (0, 0),
                               memory_space=pltpu.SMEM),
        out_shape=jax.ShapeDtypeStruct((1, 1), jnp.float32),
        scratch_shapes=[pltpu.VMEM((8, 512), jnp.float32)],
    )(output, mask)


def kernel(output, mask):
    return _tc_loss(output, mask)[0, 0]


_P4, _P3, _P2, _P1 = -0.08001088, 0.3154676, -0.67293417, 1.4373021
_P0M127 = 0.00010018903 - 127.0
_Q0, _Q1, _Q2, _Q3 = (-7.58195e-08, -2.6486185e-07, 1.3536362e-07,
                      -2.0730770e-08)


def _sc_body(out_hbm, mask_hbm, res_hbm,
             xb0, xb1, mb0, mb1, accb, sem0, sem1):
    ci = lax.axis_index("c")
    si = lax.axis_index("s")
    wid = si * 2 + ci
    xbufs = (xb0, xb1)
    mbufs = (mb0, mb1)
    sems = (sem0, sem1)

    def issue(s, p):
        g = wid * _K_SC + s
        b = (_B - _K_SC) + g // 32
        h0 = (g % 32) * 16
        ds = []
        ds.append(pltpu.async_copy(
            mask_hbm.at[b, 0, pl.ds(h0, 16), :], mbufs[p], sems[p]))
        for c in range(_C):
            src = out_hbm.at[b, c, pl.ds(h0, 16), :]
            ds.append(pltpu.async_copy(
                src, xbufs[p].at[pl.ds(c * 16, 16), :], sems[p]))
        return ds

    iot = lax.iota(jnp.int32, 16)
    zero = jnp.zeros((16,), jnp.float32)
    accs = (zero, zero, zero, zero)
    pend = issue(0, 0)
    for s in range(_STEPS):
        p = s & 1
        for d in pend:
            d.wait()
        if s + 1 < _STEPS:
            pend = issue(s + 1, p ^ 1)
        xb = xbufs[p]
        mb = mbufs[p]

        def body(v, accs, xb=xb, mb=mb):
            out = []
            for j in range(4):
                vj = v + j
                r = vj & 15
                w0 = vj & -16
                w_idx = iot + w0
                m = mb[r, pl.ds(w0, 16)]
                x = plsc.load_gather(xb, [(m << 4) + r, w_idx])
                u = plsc.bitcast(jnp.maximum(x, _LO), jnp.int32)
                ef = (u >> 23).astype(jnp.float32)
                mf = plsc.bitcast((u & 0x007FFFFF) | 0x3F800000,
                                  jnp.float32)
                rm = mf - 1.0
                pv = _P4
                for co in (_P3, _P2, _P1, _P0M127):
                    pv = pv * rm + co
                l2 = ef + pv
                mfl = m.astype(jnp.float32)
                wv = _Q3
                for co in (_Q2, _Q1, _Q0):
                    wv = wv * mfl + co
                out.append(accs[j] + l2 * wv)
            return tuple(out)

        accs = plsc.parallel_loop(0, _NVEC, step=4, unroll=1,
                                  carry=accs)(body)

    accb[...] = (accs[0] + accs[1]) + (accs[2] + accs[3])
    pltpu.sync_copy(accb, res_hbm.at[pl.ds(wid * 16, 16)])


_sc_loss = pl.kernel(
    _sc_body,
    out_type=jax.ShapeDtypeStruct((_NW * 16,), jnp.float32),
    mesh=plsc.VectorSubcoreMesh(core_axis_name="c", subcore_axis_name="s"),
    compiler_params=pltpu.CompilerParams(needs_layout_passes=False,
                                         use_tc_tiling_on_sc=True),
    scratch_types=[
        pltpu.VMEM((_C * 16, 512), jnp.float32),
        pltpu.VMEM((_C * 16, 512), jnp.float32),
        pltpu.VMEM((16, 512), jnp.int32),
        pltpu.VMEM((16, 512), jnp.int32),
        pltpu.VMEM((16,), jnp.float32),
        pltpu.SemaphoreType.DMA,
        pltpu.SemaphoreType.DMA,
    ],
)
